# Initial kernel scaffold; baseline (speedup 1.0000x reference)
import jax
import jax.numpy as jnp
from jax.experimental import pallas as pl


def kernel(A, X, W_lin, b_lin, Wq, bq, Wk, bk, Wv, bv, Wo, bo):
    raise NotImplementedError("write your pallas kernel here")


# speedup vs baseline: 30.7188x; 30.7188x over previous
import jax
import jax.numpy as jnp
import numpy as np
from jax import lax
from jax.experimental import pallas as pl
from jax.experimental.pallas import tpu as pltpu
from jax.experimental.pallas import tpu_sc as plsc

N_NODES = 10000
N_EDGES = 320000
HIDDEN = 128
NUM_HEADS = 8
HEAD_DIM = HIDDEN // NUM_HEADS

NC = 2
NS = 16
LANES = 16

HALF = HIDDEN // NC
HPC = NUM_HEADS // NC
ROW_W = HALF + LANES
EPT = N_EDGES // NS
BATCH = 80
NBATCH = EPT // BATCH
N_PAD = 10240
ROWS_PER_TILE = N_PAD // NS
ZROWS = 128

_BLK = 1000
_GRID = N_NODES // _BLK

_PERM = np.array([(j // HPC) * NUM_HEADS + c * HPC + (j % HPC)
                  for c in range(NC) for j in range(HALF)], dtype=np.int32)


def _qkv_body(x_ref, wl_ref, bl_ref, wq_ref, bq_ref, wk_ref, bk_ref,
              wv_ref, bv_ref, q_ref, k_ref, v_ref):
    x = x_ref[...]
    h = jnp.dot(x, wl_ref[...], preferred_element_type=jnp.float32) + bl_ref[...]
    scaling = HEAD_DIM ** -0.5
    q_ref[...] = (jnp.dot(h, wq_ref[...], preferred_element_type=jnp.float32)
                  + bq_ref[...]) * scaling
    k_ref[...] = jnp.dot(h, wk_ref[...], preferred_element_type=jnp.float32) + bk_ref[...]
    v_ref[...] = jnp.dot(h, wv_ref[...], preferred_element_type=jnp.float32) + bv_ref[...]


def _qkv(X, WlT, bl, WqT, bq, WkT, bk, WvT, bv):
    w_spec = pl.BlockSpec((HIDDEN, HIDDEN), lambda i: (0, 0))
    b_spec = pl.BlockSpec((1, HIDDEN), lambda i: (0, 0))
    row_spec = pl.BlockSpec((_BLK, HIDDEN), lambda i: (i, 0))
    out_sds = jax.ShapeDtypeStruct((N_NODES, HIDDEN), jnp.float32)
    return pl.pallas_call(
        _qkv_body,
        grid=(_GRID,),
        in_specs=[row_spec, w_spec, b_spec, w_spec, b_spec, w_spec, b_spec,
                  w_spec, b_spec],
        out_specs=[row_spec, row_spec, row_spec],
        out_shape=[out_sds, out_sds, out_sds],
    )(X, WlT, bl, WqT, bq, WkT, bk, WvT, bv)


def _sc_body(row_hbm, col_hbm, q_hbm, k_hbm, v_hbm, out_hbm,
             idx_r, idx_q, idx_c, qb, kb, vb, msgb, zbuf, shared, sem):
    cid = lax.axis_index("c")
    sid = lax.axis_index("s")

    zero16 = jnp.zeros((LANES,), jnp.float32)

    def _zrow(r, _):
        def _zcol(j, __):
            zbuf[r, pl.ds(j * LANES, LANES)] = zero16
            return 0
        lax.fori_loop(0, ROW_W // LANES, _zcol, 0)
        return 0
    lax.fori_loop(0, ZROWS, _zrow, 0)

    def _zcopy(t, _):
        pltpu.sync_copy(zbuf, shared.at[pl.ds(sid * ROWS_PER_TILE + t * ZROWS,
                                              ZROWS)])
        return 0
    lax.fori_loop(0, ROWS_PER_TILE // ZROWS, _zcopy, 0)
    plsc.subcore_barrier()

    perm8 = lax.iota(jnp.int32, LANES) ^ 8
    perm4 = lax.iota(jnp.int32, LANES) ^ 4
    dnums = lax.GatherDimensionNumbers(offset_dims=(),
                                       collapsed_slice_dims=(0,),
                                       start_index_map=(0,))

    def _rot(x, perm):
        return lax.gather(x, perm[:, None], dnums, slice_sizes=(1,),
                          mode=lax.GatherScatterMode.PROMISE_IN_BOUNDS)

    half_base = cid * N_NODES

    def _batch(t, _):
        ebase = sid * EPT + t * BATCH
        pltpu.sync_copy(row_hbm.at[pl.ds(ebase, BATCH)], idx_r)
        pltpu.sync_copy(col_hbm.at[pl.ds(ebase, BATCH)], idx_c)
        for j in range(BATCH // LANES):
            sl = pl.ds(j * LANES, LANES)
            idx_q[sl] = idx_r[sl] + half_base
            idx_c[sl] = idx_c[sl] + half_base
        gq = pltpu.async_copy(q_hbm.at[idx_q], qb, sem)
        gk = pltpu.async_copy(k_hbm.at[idx_c], kb, sem)
        gv = pltpu.async_copy(v_hbm.at[idx_c], vb, sem)
        gq.wait()
        gk.wait()
        gv.wait()

        def _edge(e, __):
            psum = zero16
            for i in range(HALF // LANES):
                psum = psum + (qb[e, pl.ds(i * LANES, LANES)]
                               * kb[e, pl.ds(i * LANES, LANES)])
            t1 = psum + _rot(psum, perm8)
            t2 = t1 + _rot(t1, perm4)
            ex = jnp.exp(t2)
            for i in range(HALF // LANES):
                msgb[e, pl.ds(i * LANES, LANES)] = (
                    vb[e, pl.ds(i * LANES, LANES)] * ex)
            msgb[e, pl.ds(HALF, LANES)] = ex
            return 0
        lax.fori_loop(0, BATCH, _edge, 0)

        pltpu.sync_copy(msgb, shared.at[idx_r], add=True)
        return 0
    lax.fori_loop(0, NBATCH, _batch, 0)

    plsc.subcore_barrier()
    pltpu.sync_copy(shared.at[pl.ds(sid * ROWS_PER_TILE, ROWS_PER_TILE)],
                    out_hbm.at[cid].at[pl.ds(sid * ROWS_PER_TILE,
                                             ROWS_PER_TILE)])


def _sc_edge(row, col, q, k, v):
    mesh = plsc.VectorSubcoreMesh(core_axis_name="c", subcore_axis_name="s",
                                  num_cores=NC, num_subcores=NS)
    f = pl.kernel(
        _sc_body,
        out_type=jax.ShapeDtypeStruct((NC, N_PAD, ROW_W), jnp.float32),
        mesh=mesh,
        scratch_types=[
            pltpu.VMEM((BATCH,), jnp.int32),
            pltpu.VMEM((BATCH,), jnp.int32),
            pltpu.VMEM((BATCH,), jnp.int32),
            pltpu.VMEM((BATCH, HALF), jnp.float32),
            pltpu.VMEM((BATCH, HALF), jnp.float32),
            pltpu.VMEM((BATCH, HALF), jnp.float32),
            pltpu.VMEM((BATCH, ROW_W), jnp.float32),
            pltpu.VMEM((ZROWS, ROW_W), jnp.float32),
            pltpu.VMEM_SHARED((N_PAD, ROW_W), jnp.float32),
            pltpu.SemaphoreType.DMA,
        ],
        compiler_params=pltpu.CompilerParams(use_tc_tiling_on_sc=False),
    )
    return f(row, col, q, k, v)


def _final_body(p_ref, wo_ref, bo_ref, o_ref):
    p = p_ref[...]
    n0 = p[0, :, :HALF] / (jnp.tile(p[0, :, HALF:], (1, HPC)) + 1e-9)
    n1 = p[1, :, :HALF] / (jnp.tile(p[1, :, HALF:], (1, HPC)) + 1e-9)
    nrm = jnp.concatenate([n0, n1], axis=1)
    o_ref[...] = (jnp.dot(nrm, wo_ref[...],
                          preferred_element_type=jnp.float32) + bo_ref[...])


def _final(parts, WoTp, bo):
    return pl.pallas_call(
        _final_body,
        grid=(_GRID,),
        in_specs=[pl.BlockSpec((NC, _BLK, ROW_W), lambda i: (0, i, 0)),
                  pl.BlockSpec((HIDDEN, HIDDEN), lambda i: (0, 0)),
                  pl.BlockSpec((1, HIDDEN), lambda i: (0, 0))],
        out_specs=pl.BlockSpec((_BLK, HIDDEN), lambda i: (i, 0)),
        out_shape=jax.ShapeDtypeStruct((N_NODES, HIDDEN), jnp.float32),
    )(parts, WoTp, bo)


@jax.jit
def kernel(A, X, W_lin, b_lin, Wq, bq, Wk, bk, Wv, bv, Wo, bo):
    perm = jnp.asarray(_PERM)
    row = A[0]
    col = A[1]
    q, k, v = _qkv(X, W_lin.T, b_lin.reshape(1, HIDDEN),
                   Wq.T[:, perm], bq[perm].reshape(1, HIDDEN),
                   Wk.T[:, perm], bk[perm].reshape(1, HIDDEN),
                   Wv.T[:, perm], bv[perm].reshape(1, HIDDEN))
    q2 = q.reshape(N_NODES, NC, HALF).transpose(1, 0, 2).reshape(NC * N_NODES, HALF)
    k2 = k.reshape(N_NODES, NC, HALF).transpose(1, 0, 2).reshape(NC * N_NODES, HALF)
    v2 = v.reshape(N_NODES, NC, HALF).transpose(1, 0, 2).reshape(NC * N_NODES, HALF)
    parts = _sc_edge(row, col, q2, k2, v2)
    return _final(parts, Wo.T[jnp.asarray(_PERM), :], bo.reshape(1, HIDDEN))


# speedup vs baseline: 75.9388x; 2.4721x over previous
import jax
import jax.numpy as jnp
import numpy as np
from jax import lax
from jax.experimental import pallas as pl
from jax.experimental.pallas import tpu as pltpu
from jax.experimental.pallas import tpu_sc as plsc

N_NODES = 10000
N_EDGES = 320000
HIDDEN = 128
NUM_HEADS = 8
HEAD_DIM = HIDDEN // NUM_HEADS

NC = 2
NS = 16
LANES = 16

HALF = HIDDEN // NC
HPC = NUM_HEADS // NC
ROW_W = HALF + LANES
EPT = N_EDGES // NS
BATCH = 80
NBATCH = EPT // BATCH
N_PAD = 10240
ROWS_PER_TILE = N_PAD // NS
ZROWS = 128

_BLK = 1000
_GRID = N_NODES // _BLK

_PERM = np.array([(j // HPC) * NUM_HEADS + c * HPC + (j % HPC)
                  for c in range(NC) for j in range(HALF)], dtype=np.int32)


def _qkv_body(x_ref, wl_ref, bl_ref, wq_ref, bq_ref, wk_ref, bk_ref,
              wv_ref, bv_ref, q_ref, k_ref, v_ref):
    x = x_ref[...]
    h = jnp.dot(x, wl_ref[...], preferred_element_type=jnp.float32) + bl_ref[...]
    scaling = HEAD_DIM ** -0.5
    q_ref[...] = (jnp.dot(h, wq_ref[...], preferred_element_type=jnp.float32)
                  + bq_ref[...]) * scaling
    k_ref[...] = jnp.dot(h, wk_ref[...], preferred_element_type=jnp.float32) + bk_ref[...]
    v_ref[...] = jnp.dot(h, wv_ref[...], preferred_element_type=jnp.float32) + bv_ref[...]


def _qkv(X, WlT, bl, WqT, bq, WkT, bk, WvT, bv):
    w_spec = pl.BlockSpec((HIDDEN, HIDDEN), lambda i: (0, 0))
    b_spec = pl.BlockSpec((1, HIDDEN), lambda i: (0, 0))
    row_spec = pl.BlockSpec((_BLK, HIDDEN), lambda i: (i, 0))
    out_sds = jax.ShapeDtypeStruct((N_NODES, HIDDEN), jnp.float32)
    return pl.pallas_call(
        _qkv_body,
        grid=(_GRID,),
        in_specs=[row_spec, w_spec, b_spec, w_spec, b_spec, w_spec, b_spec,
                  w_spec, b_spec],
        out_specs=[row_spec, row_spec, row_spec],
        out_shape=[out_sds, out_sds, out_sds],
    )(X, WlT, bl, WqT, bq, WkT, bk, WvT, bv)


def _sc_body(row_hbm, col_hbm, q_hbm, k_hbm, v_hbm, out_hbm,
             idx_r,
             idx_q0, idx_c0, qb0, kb0, vb0, sidx0,
             idx_q1, idx_c1, qb1, kb1, vb1, sidx1,
             msgb, zbuf, shared, gsem0, gsem1):
    cid = lax.axis_index("c")
    sid = lax.axis_index("s")

    zero16 = jnp.zeros((LANES,), jnp.float32)
    sets = ((idx_q0, idx_c0, qb0, kb0, vb0, sidx0, gsem0),
            (idx_q1, idx_c1, qb1, kb1, vb1, sidx1, gsem1))

    def _zrow(r, _):
        def _zcol(j, __):
            zbuf[r, pl.ds(j * LANES, LANES)] = zero16
            return 0
        lax.fori_loop(0, ROW_W // LANES, _zcol, 0)
        return 0
    lax.fori_loop(0, ZROWS, _zrow, 0)

    def _zcopy(t, _):
        pltpu.sync_copy(zbuf, shared.at[pl.ds(sid * ROWS_PER_TILE + t * ZROWS,
                                              ZROWS)])
        return 0
    lax.fori_loop(0, ROWS_PER_TILE // ZROWS, _zcopy, 0)
    plsc.subcore_barrier()

    perm8 = lax.iota(jnp.int32, LANES) ^ 8
    perm4 = lax.iota(jnp.int32, LANES) ^ 4
    dnums = lax.GatherDimensionNumbers(offset_dims=(),
                                       collapsed_slice_dims=(0,),
                                       start_index_map=(0,))

    def _rot(x, perm):
        return lax.gather(x, perm[:, None], dnums, slice_sizes=(1,),
                          mode=lax.GatherScatterMode.PROMISE_IN_BOUNDS)

    half_base = cid * N_NODES

    def _prefetch(t, buf):
        idx_q, idx_c, qb, kb, vb, sidx, gsem = sets[buf]
        ebase = sid * EPT + t * BATCH
        pltpu.sync_copy(row_hbm.at[pl.ds(ebase, BATCH)], idx_r)
        pltpu.sync_copy(col_hbm.at[pl.ds(ebase, BATCH)], idx_c)
        for j in range(BATCH // LANES):
            sl = pl.ds(j * LANES, LANES)
            r = idx_r[sl]
            sidx[sl] = r
            idx_q[sl] = r + half_base
            idx_c[sl] = idx_c[sl] + half_base
        pltpu.async_copy(q_hbm.at[idx_q], qb, gsem)
        pltpu.async_copy(k_hbm.at[idx_c], kb, gsem)
        pltpu.async_copy(v_hbm.at[idx_c], vb, gsem)

    def _consume(t, buf):
        idx_q, idx_c, qb, kb, vb, sidx, gsem = sets[buf]
        pltpu.make_async_copy(q_hbm.at[idx_q], qb, gsem).wait()
        pltpu.make_async_copy(k_hbm.at[idx_c], kb, gsem).wait()
        pltpu.make_async_copy(v_hbm.at[idx_c], vb, gsem).wait()

        @plsc.parallel_loop(0, BATCH, unroll=2)
        def _edge(e):
            psum = zero16
            for i in range(HALF // LANES):
                psum = psum + (qb[e, pl.ds(i * LANES, LANES)]
                               * kb[e, pl.ds(i * LANES, LANES)])
            t1 = psum + _rot(psum, perm8)
            t2 = t1 + _rot(t1, perm4)
            ex = jnp.exp(t2)
            for i in range(HALF // LANES):
                msgb[e, pl.ds(i * LANES, LANES)] = (
                    vb[e, pl.ds(i * LANES, LANES)] * ex)
            msgb[e, pl.ds(HALF, LANES)] = ex

        pltpu.sync_copy(msgb, shared.at[sidx], add=True)

    _prefetch(0, 0)

    def _pair(s, _):
        t = s * 2
        _prefetch(t + 1, 1)
        _consume(t, 0)

        @pl.when(t + 2 < NBATCH)
        def _():
            _prefetch(t + 2, 0)
        _consume(t + 1, 1)
        return 0
    lax.fori_loop(0, NBATCH // 2, _pair, 0)

    plsc.subcore_barrier()
    pltpu.sync_copy(shared.at[pl.ds(sid * ROWS_PER_TILE, ROWS_PER_TILE)],
                    out_hbm.at[cid].at[pl.ds(sid * ROWS_PER_TILE,
                                             ROWS_PER_TILE)])


def _sc_edge(row, col, q, k, v):
    mesh = plsc.VectorSubcoreMesh(core_axis_name="c", subcore_axis_name="s",
                                  num_cores=NC, num_subcores=NS)
    f = pl.kernel(
        _sc_body,
        out_type=jax.ShapeDtypeStruct((NC, N_PAD, ROW_W), jnp.float32),
        mesh=mesh,
        scratch_types=(
            [pltpu.VMEM((BATCH,), jnp.int32)]
            + 2 * [pltpu.VMEM((BATCH,), jnp.int32),
                   pltpu.VMEM((BATCH,), jnp.int32),
                   pltpu.VMEM((BATCH, HALF), jnp.float32),
                   pltpu.VMEM((BATCH, HALF), jnp.float32),
                   pltpu.VMEM((BATCH, HALF), jnp.float32),
                   pltpu.VMEM((BATCH,), jnp.int32)]
            + [pltpu.VMEM((BATCH, ROW_W), jnp.float32),
               pltpu.VMEM((ZROWS, ROW_W), jnp.float32),
               pltpu.VMEM_SHARED((N_PAD, ROW_W), jnp.float32),
               pltpu.SemaphoreType.DMA,
               pltpu.SemaphoreType.DMA]
        ),
        compiler_params=pltpu.CompilerParams(use_tc_tiling_on_sc=False),
    )
    return f(row, col, q, k, v)


def _final_body(p_ref, wo_ref, bo_ref, o_ref):
    p = p_ref[...]
    n0 = p[0, :, :HALF] / (jnp.tile(p[0, :, HALF:], (1, HPC)) + 1e-9)
    n1 = p[1, :, :HALF] / (jnp.tile(p[1, :, HALF:], (1, HPC)) + 1e-9)
    nrm = jnp.concatenate([n0, n1], axis=1)
    o_ref[...] = (jnp.dot(nrm, wo_ref[...],
                          preferred_element_type=jnp.float32) + bo_ref[...])


def _final(parts, WoTp, bo):
    return pl.pallas_call(
        _final_body,
        grid=(_GRID,),
        in_specs=[pl.BlockSpec((NC, _BLK, ROW_W), lambda i: (0, i, 0)),
                  pl.BlockSpec((HIDDEN, HIDDEN), lambda i: (0, 0)),
                  pl.BlockSpec((1, HIDDEN), lambda i: (0, 0))],
        out_specs=pl.BlockSpec((_BLK, HIDDEN), lambda i: (i, 0)),
        out_shape=jax.ShapeDtypeStruct((N_NODES, HIDDEN), jnp.float32),
    )(parts, WoTp, bo)


@jax.jit
def kernel(A, X, W_lin, b_lin, Wq, bq, Wk, bk, Wv, bv, Wo, bo):
    perm = jnp.asarray(_PERM)
    row = A[0]
    col = A[1]
    q, k, v = _qkv(X, W_lin.T, b_lin.reshape(1, HIDDEN),
                   Wq.T[:, perm], bq[perm].reshape(1, HIDDEN),
                   Wk.T[:, perm], bk[perm].reshape(1, HIDDEN),
                   Wv.T[:, perm], bv[perm].reshape(1, HIDDEN))
    q2 = q.reshape(N_NODES, NC, HALF).transpose(1, 0, 2).reshape(NC * N_NODES, HALF)
    k2 = k.reshape(N_NODES, NC, HALF).transpose(1, 0, 2).reshape(NC * N_NODES, HALF)
    v2 = v.reshape(N_NODES, NC, HALF).transpose(1, 0, 2).reshape(NC * N_NODES, HALF)
    parts = _sc_edge(row, col, q2, k2, v2)
    return _final(parts, Wo.T[jnp.asarray(_PERM), :], bo.reshape(1, HIDDEN))


# speedup vs baseline: 97.9861x; 1.2903x over previous
import jax
import jax.numpy as jnp
import numpy as np
from jax import lax
from jax.experimental import pallas as pl
from jax.experimental.pallas import tpu as pltpu
from jax.experimental.pallas import tpu_sc as plsc

N_NODES = 10000
N_EDGES = 320000
HIDDEN = 128
NUM_HEADS = 8
HEAD_DIM = HIDDEN // NUM_HEADS

NC = 2
NS = 16
LANES = 16

HALF = HIDDEN // NC
HPC = NUM_HEADS // NC
ROW_W = HALF + LANES
EPT = N_EDGES // NS
BATCH = 80
NBATCH = EPT // BATCH
N_PAD = 10240
ROWS_PER_TILE = N_PAD // NS
ZROWS = 128
IDX_CHUNK = 8

_BLK = 1000
_GRID = N_NODES // _BLK

_PERM = np.array([(j // HPC) * NUM_HEADS + c * HPC + (j % HPC)
                  for c in range(NC) for j in range(HALF)], dtype=np.int32)


def _qkv_body(x_ref, wl_ref, bl_ref, wq_ref, bq_ref, wk_ref, bk_ref,
              wv_ref, bv_ref, q_ref, k_ref, v_ref):
    x = x_ref[...]
    h = jnp.dot(x, wl_ref[...], preferred_element_type=jnp.float32) + bl_ref[...]
    scaling = HEAD_DIM ** -0.5
    q_ref[...] = (jnp.dot(h, wq_ref[...], preferred_element_type=jnp.float32)
                  + bq_ref[...]) * scaling
    k_ref[...] = jnp.dot(h, wk_ref[...], preferred_element_type=jnp.float32) + bk_ref[...]
    v_ref[...] = jnp.dot(h, wv_ref[...], preferred_element_type=jnp.float32) + bv_ref[...]


def _qkv(X, WlT, bl, WqT, bq, WkT, bk, WvT, bv):
    w_spec = pl.BlockSpec((HIDDEN, HIDDEN), lambda i: (0, 0))
    b_spec = pl.BlockSpec((1, HIDDEN), lambda i: (0, 0))
    row_spec = pl.BlockSpec((_BLK, HIDDEN), lambda i: (i, 0))
    out_sds = jax.ShapeDtypeStruct((N_NODES, HIDDEN), jnp.float32)
    return pl.pallas_call(
        _qkv_body,
        grid=(_GRID,),
        in_specs=[row_spec, w_spec, b_spec, w_spec, b_spec, w_spec, b_spec,
                  w_spec, b_spec],
        out_specs=[row_spec, row_spec, row_spec],
        out_shape=[out_sds, out_sds, out_sds],
    )(X, WlT, bl, WqT, bq, WkT, bk, WvT, bv)


def _sc_body(row_hbm, col_hbm, q_hbm, k_hbm, v_hbm, out_hbm,
             ridx, cidx,
             idx_q0, idx_c0, qb0, kb0, vb0, sidx0,
             idx_q1, idx_c1, qb1, kb1, vb1, sidx1,
             msgb, zbuf, shared, gsem0, gsem1):
    cid = lax.axis_index("c")
    sid = lax.axis_index("s")

    zero16 = jnp.zeros((LANES,), jnp.float32)
    sets = ((idx_q0, idx_c0, qb0, kb0, vb0, sidx0, gsem0),
            (idx_q1, idx_c1, qb1, kb1, vb1, sidx1, gsem1))

    def _zrow(r, _):
        def _zcol(j, __):
            zbuf[r, pl.ds(j * LANES, LANES)] = zero16
            return 0
        lax.fori_loop(0, ROW_W // LANES, _zcol, 0)
        return 0
    lax.fori_loop(0, ZROWS, _zrow, 0)

    def _zcopy(t, _):
        pltpu.sync_copy(zbuf, shared.at[pl.ds(sid * ROWS_PER_TILE + t * ZROWS,
                                              ZROWS)])
        return 0
    lax.fori_loop(0, ROWS_PER_TILE // ZROWS, _zcopy, 0)
    plsc.subcore_barrier()

    perm8 = lax.iota(jnp.int32, LANES) ^ 8
    perm4 = lax.iota(jnp.int32, LANES) ^ 4
    dnums = lax.GatherDimensionNumbers(offset_dims=(),
                                       collapsed_slice_dims=(0,),
                                       start_index_map=(0,))

    def _rot(x, perm):
        return lax.gather(x, perm[:, None], dnums, slice_sizes=(1,),
                          mode=lax.GatherScatterMode.PROMISE_IN_BOUNDS)

    half_base = cid * N_NODES

    def _prefetch(t, buf):
        idx_q, idx_c, qb, kb, vb, sidx, gsem = sets[buf]

        @pl.when(t % IDX_CHUNK == 0)
        def _():
            ebase = sid * EPT + t * BATCH
            pltpu.sync_copy(row_hbm.at[pl.ds(ebase, IDX_CHUNK * BATCH)], ridx)
            pltpu.sync_copy(col_hbm.at[pl.ds(ebase, IDX_CHUNK * BATCH)], cidx)

        off = (t % IDX_CHUNK) * BATCH
        for j in range(BATCH // LANES):
            sl = pl.ds(j * LANES, LANES)
            src = pl.ds(off + j * LANES, LANES)
            r = ridx[src]
            sidx[sl] = r
            idx_q[sl] = r + half_base
            idx_c[sl] = cidx[src] + half_base
        pltpu.async_copy(q_hbm.at[idx_q], qb, gsem)
        pltpu.async_copy(k_hbm.at[idx_c], kb, gsem)
        pltpu.async_copy(v_hbm.at[idx_c], vb, gsem)

    def _consume(t, buf):
        idx_q, idx_c, qb, kb, vb, sidx, gsem = sets[buf]
        pltpu.make_async_copy(q_hbm.at[idx_q], qb, gsem).wait()
        pltpu.make_async_copy(k_hbm.at[idx_c], kb, gsem).wait()
        pltpu.make_async_copy(v_hbm.at[idx_c], vb, gsem).wait()

        @plsc.parallel_loop(0, BATCH, unroll=4)
        def _edge(e):
            psum = zero16
            for i in range(HALF // LANES):
                psum = psum + (qb[e, pl.ds(i * LANES, LANES)]
                               * kb[e, pl.ds(i * LANES, LANES)])
            t1 = psum + _rot(psum, perm8)
            t2 = t1 + _rot(t1, perm4)
            ex = jnp.exp(t2)
            for i in range(HALF // LANES):
                msgb[e, pl.ds(i * LANES, LANES)] = (
                    vb[e, pl.ds(i * LANES, LANES)] * ex)
            msgb[e, pl.ds(HALF, LANES)] = ex

        pltpu.sync_copy(msgb, shared.at[sidx], add=True)

    _prefetch(0, 0)

    def _pair(s, _):
        t = s * 2
        _prefetch(t + 1, 1)
        _consume(t, 0)

        @pl.when(t + 2 < NBATCH)
        def _():
            _prefetch(t + 2, 0)
        _consume(t + 1, 1)
        return 0
    lax.fori_loop(0, NBATCH // 2, _pair, 0)

    plsc.subcore_barrier()
    pltpu.sync_copy(shared.at[pl.ds(sid * ROWS_PER_TILE, ROWS_PER_TILE)],
                    out_hbm.at[cid].at[pl.ds(sid * ROWS_PER_TILE,
                                             ROWS_PER_TILE)])


def _sc_edge(row, col, q, k, v):
    mesh = plsc.VectorSubcoreMesh(core_axis_name="c", subcore_axis_name="s",
                                  num_cores=NC, num_subcores=NS)
    f = pl.kernel(
        _sc_body,
        out_type=jax.ShapeDtypeStruct((NC, N_PAD, ROW_W), jnp.float32),
        mesh=mesh,
        scratch_types=(
            [pltpu.VMEM((IDX_CHUNK * BATCH,), jnp.int32),
             pltpu.VMEM((IDX_CHUNK * BATCH,), jnp.int32)]
            + 2 * [pltpu.VMEM((BATCH,), jnp.int32),
                   pltpu.VMEM((BATCH,), jnp.int32),
                   pltpu.VMEM((BATCH, HALF), jnp.float32),
                   pltpu.VMEM((BATCH, HALF), jnp.float32),
                   pltpu.VMEM((BATCH, HALF), jnp.float32),
                   pltpu.VMEM((BATCH,), jnp.int32)]
            + [pltpu.VMEM((BATCH, ROW_W), jnp.float32),
               pltpu.VMEM((ZROWS, ROW_W), jnp.float32),
               pltpu.VMEM_SHARED((N_PAD, ROW_W), jnp.float32),
               pltpu.SemaphoreType.DMA,
               pltpu.SemaphoreType.DMA]
        ),
        compiler_params=pltpu.CompilerParams(use_tc_tiling_on_sc=False),
    )
    return f(row, col, q, k, v)


def _final_body(p_ref, wo_ref, bo_ref, o_ref):
    p = p_ref[...]
    n0 = p[0, :, :HALF] / (jnp.tile(p[0, :, HALF:], (1, HPC)) + 1e-9)
    n1 = p[1, :, :HALF] / (jnp.tile(p[1, :, HALF:], (1, HPC)) + 1e-9)
    nrm = jnp.concatenate([n0, n1], axis=1)
    o_ref[...] = (jnp.dot(nrm, wo_ref[...],
                          preferred_element_type=jnp.float32) + bo_ref[...])


def _final(parts, WoTp, bo):
    return pl.pallas_call(
        _final_body,
        grid=(_GRID,),
        in_specs=[pl.BlockSpec((NC, _BLK, ROW_W), lambda i: (0, i, 0)),
                  pl.BlockSpec((HIDDEN, HIDDEN), lambda i: (0, 0)),
                  pl.BlockSpec((1, HIDDEN), lambda i: (0, 0))],
        out_specs=pl.BlockSpec((_BLK, HIDDEN), lambda i: (i, 0)),
        out_shape=jax.ShapeDtypeStruct((N_NODES, HIDDEN), jnp.float32),
    )(parts, WoTp, bo)


@jax.jit
def kernel(A, X, W_lin, b_lin, Wq, bq, Wk, bk, Wv, bv, Wo, bo):
    perm = jnp.asarray(_PERM)
    row = A[0]
    col = A[1]
    q, k, v = _qkv(X, W_lin.T, b_lin.reshape(1, HIDDEN),
                   Wq.T[:, perm], bq[perm].reshape(1, HIDDEN),
                   Wk.T[:, perm], bk[perm].reshape(1, HIDDEN),
                   Wv.T[:, perm], bv[perm].reshape(1, HIDDEN))
    q2 = q.reshape(N_NODES, NC, HALF).transpose(1, 0, 2).reshape(NC * N_NODES, HALF)
    k2 = k.reshape(N_NODES, NC, HALF).transpose(1, 0, 2).reshape(NC * N_NODES, HALF)
    v2 = v.reshape(N_NODES, NC, HALF).transpose(1, 0, 2).reshape(NC * N_NODES, HALF)
    parts = _sc_edge(row, col, q2, k2, v2)
    return _final(parts, Wo.T[jnp.asarray(_PERM), :], bo.reshape(1, HIDDEN))


# speedup vs baseline: 105.1279x; 1.0729x over previous
import jax
import jax.numpy as jnp
import numpy as np
from jax import lax
from jax.experimental import pallas as pl
from jax.experimental.pallas import tpu as pltpu
from jax.experimental.pallas import tpu_sc as plsc

N_NODES = 10000
N_EDGES = 320000
HIDDEN = 128
NUM_HEADS = 8
HEAD_DIM = HIDDEN // NUM_HEADS

NC = 2
NS = 16
LANES = 16

HALF = HIDDEN // NC
HPC = NUM_HEADS // NC
ROW_W = HALF + LANES
EPT = N_EDGES // NS
BATCH = 80
NBATCH = EPT // BATCH
N_PAD = 10240
ROWS_PER_TILE = N_PAD // NS
ZROWS = 128
IDX_CHUNK = 8

_BLK = 1000
_GRID = N_NODES // _BLK

_PERM = np.array([(j // HPC) * NUM_HEADS + c * HPC + (j % HPC)
                  for c in range(NC) for j in range(HALF)], dtype=np.int32)


def _qkv_body(x_ref, wl_ref, bl_ref, wq_ref, bq_ref, wk_ref, bk_ref,
              wv_ref, bv_ref, q_ref, k_ref, v_ref):
    x = x_ref[...]
    h = jnp.dot(x, wl_ref[...], preferred_element_type=jnp.float32) + bl_ref[...]
    scaling = HEAD_DIM ** -0.5
    q_ref[...] = (jnp.dot(h, wq_ref[...], preferred_element_type=jnp.float32)
                  + bq_ref[...]) * scaling
    k_ref[...] = jnp.dot(h, wk_ref[...], preferred_element_type=jnp.float32) + bk_ref[...]
    v_ref[...] = jnp.dot(h, wv_ref[...], preferred_element_type=jnp.float32) + bv_ref[...]


def _qkv(X, WlT, bl, WqT, bq, WkT, bk, WvT, bv):
    w_spec = pl.BlockSpec((HIDDEN, HIDDEN), lambda i: (0, 0))
    b_spec = pl.BlockSpec((1, HIDDEN), lambda i: (0, 0))
    row_spec = pl.BlockSpec((_BLK, HIDDEN), lambda i: (i, 0))
    out_sds = jax.ShapeDtypeStruct((N_NODES, HIDDEN), jnp.float32)
    return pl.pallas_call(
        _qkv_body,
        grid=(_GRID,),
        in_specs=[row_spec, w_spec, b_spec, w_spec, b_spec, w_spec, b_spec,
                  w_spec, b_spec],
        out_specs=[row_spec, row_spec, row_spec],
        out_shape=[out_sds, out_sds, out_sds],
    )(X, WlT, bl, WqT, bq, WkT, bk, WvT, bv)


def _sc_body(row_hbm, col_hbm, q_hbm, k_hbm, v_hbm, out_hbm,
             ridx, cidx,
             idx_q0, idx_c0, qb0, kb0, vb0, sidx0,
             idx_q1, idx_c1, qb1, kb1, vb1, sidx1,
             msgb0, msgb1, ssidx0, ssidx1, zbuf, shared, gsem0, gsem1, ssem):
    cid = lax.axis_index("c")
    sid = lax.axis_index("s")

    zero16 = jnp.zeros((LANES,), jnp.float32)
    sets = ((idx_q0, idx_c0, qb0, kb0, vb0, sidx0, gsem0),
            (idx_q1, idx_c1, qb1, kb1, vb1, sidx1, gsem1))
    scat = ((msgb0, ssidx0), (msgb1, ssidx1))

    def _zrow(r, _):
        def _zcol(j, __):
            zbuf[r, pl.ds(j * LANES, LANES)] = zero16
            return 0
        lax.fori_loop(0, ROW_W // LANES, _zcol, 0)
        return 0
    lax.fori_loop(0, ZROWS, _zrow, 0)

    def _zcopy(t, _):
        pltpu.sync_copy(zbuf, shared.at[pl.ds(sid * ROWS_PER_TILE + t * ZROWS,
                                              ZROWS)])
        return 0
    lax.fori_loop(0, ROWS_PER_TILE // ZROWS, _zcopy, 0)
    plsc.subcore_barrier()

    perm8 = lax.iota(jnp.int32, LANES) ^ 8
    perm4 = lax.iota(jnp.int32, LANES) ^ 4
    dnums = lax.GatherDimensionNumbers(offset_dims=(),
                                       collapsed_slice_dims=(0,),
                                       start_index_map=(0,))

    def _rot(x, perm):
        return lax.gather(x, perm[:, None], dnums, slice_sizes=(1,),
                          mode=lax.GatherScatterMode.PROMISE_IN_BOUNDS)

    half_base = cid * N_NODES

    def _prefetch(t, buf):
        idx_q, idx_c, qb, kb, vb, sidx, gsem = sets[buf]

        @pl.when(t % IDX_CHUNK == 0)
        def _():
            ebase = sid * EPT + t * BATCH
            pltpu.sync_copy(row_hbm.at[pl.ds(ebase, IDX_CHUNK * BATCH)], ridx)
            pltpu.sync_copy(col_hbm.at[pl.ds(ebase, IDX_CHUNK * BATCH)], cidx)

        off = (t % IDX_CHUNK) * BATCH
        for j in range(BATCH // LANES):
            sl = pl.ds(j * LANES, LANES)
            src = pl.ds(off + j * LANES, LANES)
            r = ridx[src]
            sidx[sl] = r
            idx_q[sl] = r + half_base
            idx_c[sl] = cidx[src] + half_base
        pltpu.async_copy(q_hbm.at[idx_q], qb, gsem)
        pltpu.async_copy(k_hbm.at[idx_c], kb, gsem)
        pltpu.async_copy(v_hbm.at[idx_c], vb, gsem)

    def _consume(t, buf):
        idx_q, idx_c, qb, kb, vb, sidx, gsem = sets[buf]
        msgb, ssidx = scat[buf]
        pltpu.make_async_copy(q_hbm.at[idx_q], qb, gsem).wait()
        pltpu.make_async_copy(k_hbm.at[idx_c], kb, gsem).wait()
        pltpu.make_async_copy(v_hbm.at[idx_c], vb, gsem).wait()

        @pl.when(t >= 2)
        def _():
            pltpu.make_async_copy(msgb, shared.at[ssidx], ssem).wait()

        for j in range(BATCH // LANES):
            sl = pl.ds(j * LANES, LANES)
            ssidx[sl] = sidx[sl]

        @plsc.parallel_loop(0, BATCH, unroll=4)
        def _edge(e):
            psum = zero16
            for i in range(HALF // LANES):
                psum = psum + (qb[e, pl.ds(i * LANES, LANES)]
                               * kb[e, pl.ds(i * LANES, LANES)])
            t1 = psum + _rot(psum, perm8)
            t2 = t1 + _rot(t1, perm4)
            ex = jnp.exp(t2)
            for i in range(HALF // LANES):
                msgb[e, pl.ds(i * LANES, LANES)] = (
                    vb[e, pl.ds(i * LANES, LANES)] * ex)
            msgb[e, pl.ds(HALF, LANES)] = ex

        pltpu.async_copy(msgb, shared.at[ssidx], ssem, add=True)

    _prefetch(0, 0)

    def _pair(s, _):
        t = s * 2
        _prefetch(t + 1, 1)
        _consume(t, 0)

        @pl.when(t + 2 < NBATCH)
        def _():
            _prefetch(t + 2, 0)
        _consume(t + 1, 1)
        return 0
    lax.fori_loop(0, NBATCH // 2, _pair, 0)

    pltpu.make_async_copy(scat[0][0], shared.at[scat[0][1]], ssem).wait()
    pltpu.make_async_copy(scat[1][0], shared.at[scat[1][1]], ssem).wait()

    plsc.subcore_barrier()
    pltpu.sync_copy(shared.at[pl.ds(sid * ROWS_PER_TILE, ROWS_PER_TILE)],
                    out_hbm.at[cid].at[pl.ds(sid * ROWS_PER_TILE,
                                             ROWS_PER_TILE)])


def _sc_edge(row, col, q, k, v):
    mesh = plsc.VectorSubcoreMesh(core_axis_name="c", subcore_axis_name="s",
                                  num_cores=NC, num_subcores=NS)
    f = pl.kernel(
        _sc_body,
        out_type=jax.ShapeDtypeStruct((NC, N_PAD, ROW_W), jnp.float32),
        mesh=mesh,
        scratch_types=(
            [pltpu.VMEM((IDX_CHUNK * BATCH,), jnp.int32),
             pltpu.VMEM((IDX_CHUNK * BATCH,), jnp.int32)]
            + 2 * [pltpu.VMEM((BATCH,), jnp.int32),
                   pltpu.VMEM((BATCH,), jnp.int32),
                   pltpu.VMEM((BATCH, HALF), jnp.float32),
                   pltpu.VMEM((BATCH, HALF), jnp.float32),
                   pltpu.VMEM((BATCH, HALF), jnp.float32),
                   pltpu.VMEM((BATCH,), jnp.int32)]
            + [pltpu.VMEM((BATCH, ROW_W), jnp.float32),
               pltpu.VMEM((BATCH, ROW_W), jnp.float32),
               pltpu.VMEM((BATCH,), jnp.int32),
               pltpu.VMEM((BATCH,), jnp.int32),
               pltpu.VMEM((ZROWS, ROW_W), jnp.float32),
               pltpu.VMEM_SHARED((N_PAD, ROW_W), jnp.float32),
               pltpu.SemaphoreType.DMA,
               pltpu.SemaphoreType.DMA,
               pltpu.SemaphoreType.DMA]
        ),
        compiler_params=pltpu.CompilerParams(use_tc_tiling_on_sc=False),
    )
    return f(row, col, q, k, v)


def _final_body(p_ref, wo_ref, bo_ref, o_ref):
    p = p_ref[...]
    n0 = p[0, :, :HALF] / (jnp.tile(p[0, :, HALF:], (1, HPC)) + 1e-9)
    n1 = p[1, :, :HALF] / (jnp.tile(p[1, :, HALF:], (1, HPC)) + 1e-9)
    nrm = jnp.concatenate([n0, n1], axis=1)
    o_ref[...] = (jnp.dot(nrm, wo_ref[...],
                          preferred_element_type=jnp.float32) + bo_ref[...])


def _final(parts, WoTp, bo):
    return pl.pallas_call(
        _final_body,
        grid=(_GRID,),
        in_specs=[pl.BlockSpec((NC, _BLK, ROW_W), lambda i: (0, i, 0)),
                  pl.BlockSpec((HIDDEN, HIDDEN), lambda i: (0, 0)),
                  pl.BlockSpec((1, HIDDEN), lambda i: (0, 0))],
        out_specs=pl.BlockSpec((_BLK, HIDDEN), lambda i: (i, 0)),
        out_shape=jax.ShapeDtypeStruct((N_NODES, HIDDEN), jnp.float32),
    )(parts, WoTp, bo)


@jax.jit
def kernel(A, X, W_lin, b_lin, Wq, bq, Wk, bk, Wv, bv, Wo, bo):
    perm = jnp.asarray(_PERM)
    row = A[0]
    col = A[1]
    q, k, v = _qkv(X, W_lin.T, b_lin.reshape(1, HIDDEN),
                   Wq.T[:, perm], bq[perm].reshape(1, HIDDEN),
                   Wk.T[:, perm], bk[perm].reshape(1, HIDDEN),
                   Wv.T[:, perm], bv[perm].reshape(1, HIDDEN))
    q2 = q.reshape(N_NODES, NC, HALF).transpose(1, 0, 2).reshape(NC * N_NODES, HALF)
    k2 = k.reshape(N_NODES, NC, HALF).transpose(1, 0, 2).reshape(NC * N_NODES, HALF)
    v2 = v.reshape(N_NODES, NC, HALF).transpose(1, 0, 2).reshape(NC * N_NODES, HALF)
    parts = _sc_edge(row, col, q2, k2, v2)
    return _final(parts, Wo.T[jnp.asarray(_PERM), :], bo.reshape(1, HIDDEN))
